# baseline (speedup 1.0000x reference)
import functools

import jax
import jax.numpy as jnp
from jax import lax
from jax.experimental import pallas as pl
from jax.experimental.pallas import tpu as pltpu
from jax.experimental.pallas import tpu_sc as plsc

B = 16384
S = 200
D = 64
N_CORES = 2
N_SUBCORES = 16
NW = N_CORES * N_SUBCORES
RPW = B // NW
C = 1
NBUF = 8
STEPS = RPW // C
HALF = S // 2
NG = 2 * C
LANES = 16
DV = D // LANES


def _sc_pool(x1, table, row_off, nrows):
  rpw = nrows // NW
  steps = rpw // C
  mesh = plsc.VectorSubcoreMesh(core_axis_name="c", subcore_axis_name="s")

  @functools.partial(
      pl.kernel,
      mesh=mesh,
      out_type=jax.ShapeDtypeStruct((nrows, D), jnp.float32),
      compiler_params=pltpu.CompilerParams(use_tc_tiling_on_sc=False),
      scratch_types=(
          [pltpu.VMEM((NG * HALF,), jnp.int32) for _ in range(NBUF)] +
          [pltpu.VMEM((C * S, D), jnp.float32) for _ in range(NBUF)] +
          [pltpu.VMEM((C, D), jnp.float32)] +
          [pltpu.SemaphoreType.DMA for _ in range(2 * NBUF)]
      ),
  )
  def k(x1_hbm, table_hbm, out_hbm, *scr):
    idx = scr[:NBUF]
    rows = scr[NBUF:2 * NBUF]
    pool_v = scr[2 * NBUF]
    isem = scr[2 * NBUF + 1:2 * NBUF + 1 + NBUF]
    rsem = scr[2 * NBUF + 1 + NBUF:]
    cid = lax.axis_index("c")
    sid = lax.axis_index("s")
    wid = sid * N_CORES + cid
    base = row_off + wid * rpw

    def fire_gathers(nb):
      pltpu.async_copy(table_hbm.at[idx[nb]], rows[nb], rsem[nb])

    def drain_gathers(b):
      pltpu.make_async_copy(table_hbm.at[idx[b]], rows[b], rsem[b]).wait()

    def fire_idx(u, b):
      pltpu.async_copy(
          x1_hbm.at[pl.ds(S * (base + u * C), NG * HALF)], idx[b], isem[b])

    def drain_idx(b):
      pltpu.make_async_copy(
          x1_hbm.at[pl.ds(0, NG * HALF)], idx[b], isem[b]).wait()

    def compute(t, b):
      row0 = base + t * C
      for r in range(C):
        def body(i, accs, r=r, b=b):
          out = []
          for d in range(DV):
            a = accs[d]
            for u in range(4):
              a = jnp.maximum(
                  a, rows[b][r * S + i * 4 + u, pl.ds(d * LANES, LANES)])
            out.append(a)
          return tuple(out)
        neg = jnp.full((LANES,), -jnp.inf, jnp.float32)
        accs = lax.fori_loop(0, S // 4, body, (neg,) * DV)
        for d in range(DV):
          pool_v[r, pl.ds(d * LANES, LANES)] = accs[d]
      pltpu.sync_copy(pool_v, out_hbm.at[pl.ds(row0 - row_off, C)])

    def phase(t, b):
      fb = (b + NBUF - 1) % NBUF

      @pl.when(t + NBUF - 1 < steps)
      def _():
        drain_idx(fb)
        fire_gathers(fb)

      drain_gathers(b)

      @pl.when(t + NBUF < steps)
      def _():
        fire_idx(t + NBUF, b)

      compute(t, b)

    pltpu.sync_copy(x1_hbm.at[pl.ds(S * base, NG * HALF)], idx[0])
    fire_gathers(0)
    for u in range(1, NBUF):
      fire_idx(u, u)
    for u in range(1, NBUF - 1):
      drain_idx(u)
      fire_gathers(u)

    def outer(i, carry):
      for p in range(NBUF):
        phase(NBUF * i + p, p)
      return carry

    lax.fori_loop(0, steps // NBUF, outer, 0)

  return k(x1, table)


def _matmul(pooled, W, b2):
  N = W.shape[0]
  M = pooled.shape[0]
  BM = 1024

  def mm(p_ref, w_ref, b_ref, o_ref):
    o_ref[...] = lax.dot_general(
        p_ref[...], w_ref[...], (((1,), (1,)), ((), ())),
        preferred_element_type=jnp.float32) + b_ref[...]

  return pl.pallas_call(
      mm,
      grid=(M // BM,),
      in_specs=[
          pl.BlockSpec((BM, D), lambda i: (i, 0)),
          pl.BlockSpec((N, D), lambda i: (0, 0)),
          pl.BlockSpec((1, N), lambda i: (0, 0)),
      ],
      out_specs=pl.BlockSpec((BM, N), lambda i: (i, 0)),
      out_shape=jax.ShapeDtypeStruct((M, N), jnp.float32),
  )(pooled, W, b2)


NCH = 1


def kernel(x, table, W, b):
  x1 = x.astype(jnp.int32).reshape(B * S)
  b2 = b.reshape(1, -1)
  bc = B // NCH
  outs = []
  for i in range(NCH):
    pooled = _sc_pool(x1, table, i * bc, bc)
    outs.append(_matmul(pooled, W, b2))
  return jnp.concatenate(outs, axis=0)


# speedup vs baseline: 1.0502x; 1.0502x over previous
import functools

import jax
import jax.numpy as jnp
from jax import lax
from jax.experimental import pallas as pl
from jax.experimental.pallas import tpu as pltpu
from jax.experimental.pallas import tpu_sc as plsc

B = 16384
S = 200
D = 64
N_CORES = 2
N_SUBCORES = 16
NW = N_CORES * N_SUBCORES
RPW = B // NW
C = 2
NBUF = 4
STEPS = RPW // C
HALF = S // 2
NG = 2 * C
LANES = 16
DV = D // LANES


def _sc_pool(x1, table, row_off, nrows):
  rpw = nrows // NW
  steps = rpw // C
  mesh = plsc.VectorSubcoreMesh(core_axis_name="c", subcore_axis_name="s")

  @functools.partial(
      pl.kernel,
      mesh=mesh,
      out_type=jax.ShapeDtypeStruct((nrows, D), jnp.float32),
      compiler_params=pltpu.CompilerParams(use_tc_tiling_on_sc=False),
      scratch_types=(
          [pltpu.VMEM((NG * HALF,), jnp.int32) for _ in range(NBUF)] +
          [pltpu.VMEM((C * S, D), jnp.float32) for _ in range(NBUF)] +
          [pltpu.VMEM((C, D), jnp.float32)] +
          [pltpu.SemaphoreType.DMA for _ in range(2 * NBUF)]
      ),
  )
  def k(x1_hbm, table_hbm, out_hbm, *scr):
    idx = scr[:NBUF]
    rows = scr[NBUF:2 * NBUF]
    pool_v = scr[2 * NBUF]
    isem = scr[2 * NBUF + 1:2 * NBUF + 1 + NBUF]
    rsem = scr[2 * NBUF + 1 + NBUF:]
    cid = lax.axis_index("c")
    sid = lax.axis_index("s")
    wid = sid * N_CORES + cid
    base = row_off + wid * rpw

    def fire_gathers(nb):
      pltpu.async_copy(table_hbm.at[idx[nb]], rows[nb], rsem[nb])

    def drain_gathers(b):
      pltpu.make_async_copy(table_hbm.at[idx[b]], rows[b], rsem[b]).wait()

    def fire_idx(u, b):
      pltpu.async_copy(
          x1_hbm.at[pl.ds(S * (base + u * C), NG * HALF)], idx[b], isem[b])

    def drain_idx(b):
      pltpu.make_async_copy(
          x1_hbm.at[pl.ds(0, NG * HALF)], idx[b], isem[b]).wait()

    def compute(t, b):
      row0 = base + t * C
      for r in range(C):
        def body(i, accs, r=r, b=b):
          out = []
          for d in range(DV):
            a = accs[d]
            for u in range(4):
              a = jnp.maximum(
                  a, rows[b][r * S + i * 4 + u, pl.ds(d * LANES, LANES)])
            out.append(a)
          return tuple(out)
        neg = jnp.full((LANES,), -jnp.inf, jnp.float32)
        accs = lax.fori_loop(0, S // 4, body, (neg,) * DV)
        for d in range(DV):
          pool_v[r, pl.ds(d * LANES, LANES)] = accs[d]
      pltpu.sync_copy(pool_v, out_hbm.at[pl.ds(row0 - row_off, C)])

    def phase(t, b):
      fb = (b + NBUF - 1) % NBUF

      @pl.when(t + NBUF - 1 < steps)
      def _():
        drain_idx(fb)
        fire_gathers(fb)

      drain_gathers(b)

      @pl.when(t + NBUF < steps)
      def _():
        fire_idx(t + NBUF, b)

      compute(t, b)

    pltpu.sync_copy(x1_hbm.at[pl.ds(S * base, NG * HALF)], idx[0])
    fire_gathers(0)
    for u in range(1, NBUF):
      fire_idx(u, u)
    for u in range(1, NBUF - 1):
      drain_idx(u)
      fire_gathers(u)

    def outer(i, carry):
      for p in range(NBUF):
        phase(NBUF * i + p, p)
      return carry

    lax.fori_loop(0, steps // NBUF, outer, 0)

  return k(x1, table)


def _matmul(pooled, W, b2):
  N = W.shape[0]
  M = pooled.shape[0]
  BM = 2048

  def mm(p_ref, w_ref, b_ref, o_ref):
    o_ref[...] = lax.dot_general(
        p_ref[...], w_ref[...], (((1,), (1,)), ((), ())),
        preferred_element_type=jnp.float32) + b_ref[...]

  return pl.pallas_call(
      mm,
      grid=(M // BM,),
      in_specs=[
          pl.BlockSpec((BM, D), lambda i: (i, 0)),
          pl.BlockSpec((N, D), lambda i: (0, 0)),
          pl.BlockSpec((1, N), lambda i: (0, 0)),
      ],
      out_specs=pl.BlockSpec((BM, N), lambda i: (i, 0)),
      out_shape=jax.ShapeDtypeStruct((M, N), jnp.float32),
  )(pooled, W, b2)


NCH = 1


def kernel(x, table, W, b):
  x1 = x.astype(jnp.int32).reshape(B * S)
  b2 = b.reshape(1, -1)
  bc = B // NCH
  outs = []
  for i in range(NCH):
    pooled = _sc_pool(x1, table, i * bc, bc)
    outs.append(_matmul(pooled, W, b2))
  return jnp.concatenate(outs, axis=0)


# speedup vs baseline: 1.0507x; 1.0005x over previous
import functools

import jax
import jax.numpy as jnp
from jax import lax
from jax.experimental import pallas as pl
from jax.experimental.pallas import tpu as pltpu
from jax.experimental.pallas import tpu_sc as plsc

B = 16384
S = 200
D = 64
N_CORES = 2
N_SUBCORES = 16
NW = N_CORES * N_SUBCORES
RPW = B // NW
C = 2
NBUF = 4
STEPS = RPW // C
HALF = S // 2
NG = 2 * C
LANES = 16
DV = D // LANES


def _sc_pool(x1, table, row_off, nrows):
  rpw = nrows // NW
  steps = rpw // C
  mesh = plsc.VectorSubcoreMesh(core_axis_name="c", subcore_axis_name="s")

  @functools.partial(
      pl.kernel,
      mesh=mesh,
      out_type=jax.ShapeDtypeStruct((nrows, D), jnp.float32),
      compiler_params=pltpu.CompilerParams(use_tc_tiling_on_sc=False),
      scratch_types=(
          [pltpu.VMEM((NG * HALF,), jnp.int32) for _ in range(NBUF)] +
          [pltpu.VMEM((C * S, D), jnp.float32) for _ in range(NBUF)] +
          [pltpu.VMEM((C, D), jnp.float32)] +
          [pltpu.SemaphoreType.DMA for _ in range(2 * NBUF)]
      ),
  )
  def k(x1_hbm, table_hbm, out_hbm, *scr):
    idx = scr[:NBUF]
    rows = scr[NBUF:2 * NBUF]
    pool_v = scr[2 * NBUF]
    isem = scr[2 * NBUF + 1:2 * NBUF + 1 + NBUF]
    rsem = scr[2 * NBUF + 1 + NBUF:]
    cid = lax.axis_index("c")
    sid = lax.axis_index("s")
    wid = sid * N_CORES + cid
    base = row_off + wid * rpw

    def fire_gathers(nb):
      pltpu.async_copy(table_hbm.at[idx[nb]], rows[nb], rsem[nb])

    def drain_gathers(b):
      pltpu.make_async_copy(table_hbm.at[idx[b]], rows[b], rsem[b]).wait()

    def fire_idx(u, b):
      pltpu.async_copy(
          x1_hbm.at[pl.ds(S * (base + u * C), NG * HALF)], idx[b], isem[b])

    def drain_idx(b):
      pltpu.make_async_copy(
          x1_hbm.at[pl.ds(0, NG * HALF)], idx[b], isem[b]).wait()

    def compute(t, b):
      row0 = base + t * C
      for r in range(C):
        def body(i, accs, r=r, b=b):
          out = []
          for d in range(DV):
            a = accs[d]
            for u in range(4):
              a = jnp.maximum(
                  a, rows[b][r * S + i * 4 + u, pl.ds(d * LANES, LANES)])
            out.append(a)
          return tuple(out)
        neg = jnp.full((LANES,), -jnp.inf, jnp.float32)
        accs = lax.fori_loop(0, S // 4, body, (neg,) * DV)
        for d in range(DV):
          pool_v[r, pl.ds(d * LANES, LANES)] = accs[d]
      pltpu.sync_copy(pool_v, out_hbm.at[pl.ds(row0 - row_off, C)])

    def phase(t, b):
      fb = (b + NBUF - 1) % NBUF

      @pl.when(t + NBUF - 1 < steps)
      def _():
        drain_idx(fb)
        fire_gathers(fb)

      drain_gathers(b)

      @pl.when(t + NBUF < steps)
      def _():
        fire_idx(t + NBUF, b)

      compute(t, b)

    pltpu.sync_copy(x1_hbm.at[pl.ds(S * base, NG * HALF)], idx[0])
    fire_gathers(0)
    for u in range(1, NBUF):
      fire_idx(u, u)
    for u in range(1, NBUF - 1):
      drain_idx(u)
      fire_gathers(u)

    def outer(i, carry):
      for p in range(NBUF):
        phase(NBUF * i + p, p)
      return carry

    lax.fori_loop(0, steps // NBUF, outer, 0)

  return k(x1, table)


def _matmul(pooled, W, b2):
  N = W.shape[0]
  M = pooled.shape[0]
  BM = 2048

  def mm(p_ref, w_ref, b_ref, o_ref):
    o_ref[...] = lax.dot_general(
        p_ref[...], w_ref[...], (((1,), (1,)), ((), ())),
        precision=lax.Precision.DEFAULT,
        preferred_element_type=jnp.float32) + b_ref[...]

  return pl.pallas_call(
      mm,
      grid=(M // BM,),
      in_specs=[
          pl.BlockSpec((BM, D), lambda i: (i, 0)),
          pl.BlockSpec((N, D), lambda i: (0, 0)),
          pl.BlockSpec((1, N), lambda i: (0, 0)),
      ],
      out_specs=pl.BlockSpec((BM, N), lambda i: (i, 0)),
      out_shape=jax.ShapeDtypeStruct((M, N), jnp.float32),
  )(pooled, W, b2)


NCH = 1


def kernel(x, table, W, b):
  x1 = x.astype(jnp.int32).reshape(B * S)
  b2 = b.reshape(1, -1)
  bc = B // NCH
  outs = []
  for i in range(NCH):
    pooled = _sc_pool(x1, table, i * bc, bc)
    outs.append(_matmul(pooled, W, b2))
  return jnp.concatenate(outs, axis=0)
